# baseline (speedup 1.0000x reference)
import functools

import jax
import jax.numpy as jnp
from jax import lax
from jax.experimental import pallas as pl
from jax.experimental.pallas import tpu as pltpu
from jax.experimental.pallas import tpu_sc as plsc

_NC = 2
_NS = 16
_NW = _NC * _NS
_CH = 80


def _tc_matmul(x, w_ext, bias_row):
    n, d = x.shape
    dw = w_ext.shape[1]
    blk = 400

    def body(x_ref, w_ref, b_ref, o_ref):
        o_ref[...] = (
            jnp.dot(x_ref[...], w_ref[...], preferred_element_type=jnp.float32)
            + b_ref[...]
        )

    return pl.pallas_call(
        body,
        grid=(n // blk,),
        in_specs=[
            pl.BlockSpec((blk, d), lambda i: (i, 0)),
            pl.BlockSpec((d, dw), lambda i: (0, 0)),
            pl.BlockSpec((1, dw), lambda i: (0, 0)),
        ],
        out_specs=pl.BlockSpec((blk, dw), lambda i: (i, 0)),
        out_shape=jax.ShapeDtypeStruct((n, dw), jnp.float32),
    )(x, w_ext, bias_row)


def _row_partition(n_rows, n_tiles, max_chunk):
    per = -(-n_rows // n_tiles)
    per = ((per + 7) // 8) * 8
    parts = []
    base = 0
    for _ in range(n_tiles):
        cnt = max(0, min(per, n_rows - base))
        sizes = []
        left = cnt
        while left > 0:
            sz = min(max_chunk, left)
            sizes.append(sz)
            left -= sz
        parts.append((base, sizes))
        base += cnt
    return parts


def _sc_edge_weights(src3, dst3, a, b):
    _, rpw, ch = src3.shape
    n = a.shape[0]
    seg = 2000
    mesh = plsc.VectorSubcoreMesh(core_axis_name="c", subcore_axis_name="s")

    @functools.partial(
        pl.kernel,
        out_type=(
            jax.ShapeDtypeStruct((_NW, rpw, ch), jnp.float32),
            jax.ShapeDtypeStruct((_NC * n,), jnp.float32),
        ),
        mesh=mesh,
        scratch_types=(
            pltpu.VMEM((n,), jnp.float32),
            pltpu.VMEM((n,), jnp.float32),
            pltpu.VMEM((rpw, ch), jnp.int32),
            pltpu.VMEM((rpw, ch), jnp.int32),
            pltpu.VMEM((rpw, ch), jnp.float32),
            pltpu.VMEM((seg,), jnp.float32),
            pltpu.VMEM_SHARED((n,), jnp.float32),
            pltpu.SemaphoreType.DMA,
        ),
        compiler_params=pltpu.CompilerParams(needs_layout_passes=False),
    )
    def kern(src_hbm, dst_hbm, a_hbm, b_hbm, ew_hbm, degp_hbm,
             a_v, b_v, src_v, dst_v, ew_v, stage_v, deg_sh, sem):
        cid = lax.axis_index("c")
        sid = lax.axis_index("s")
        wid = cid * _NS + sid

        @pl.when(sid < n // seg)
        def _():
            for t in range(seg // 16):
                stage_v[pl.ds(t * 16, 16)] = jnp.zeros((16,), jnp.float32)
            pltpu.sync_copy(stage_v, deg_sh.at[pl.ds(sid * seg, seg)])

        pltpu.sync_copy(a_hbm, a_v)
        pltpu.sync_copy(b_hbm, b_v)
        pltpu.sync_copy(src_hbm.at[wid], src_v)
        pltpu.sync_copy(dst_hbm.at[wid], dst_v)
        plsc.subcore_barrier()

        def chunk(r_):
            for g in range(ch // 16):
                sv = src_v[r_, pl.ds(g * 16, 16)]
                dv = dst_v[r_, pl.ds(g * 16, 16)]
                av = plsc.load_gather(a_v, [sv])
                bv = plsc.load_gather(b_v, [dv])
                ew = 1.0 / (1.0 + jnp.exp(-(av + bv)))
                ew_v[r_, pl.ds(g * 16, 16)] = ew

        pl.loop(0, rpw)(chunk)

        def fire(r0):
            descs = [
                pltpu.async_copy(
                    ew_v.at[r0 + j], deg_sh.at[dst_v.at[r0 + j]], sem, add=True
                )
                for j in range(25)
            ]
            for de in descs:
                de.wait()

        pl.loop(0, rpw, step=25)(fire)

        pltpu.sync_copy(ew_v, ew_hbm.at[wid])
        plsc.subcore_barrier()

        @pl.when(sid < n // seg)
        def _():
            pltpu.sync_copy(deg_sh.at[pl.ds(sid * seg, seg)], stage_v)
            pltpu.sync_copy(
                stage_v, degp_hbm.at[pl.ds(cid * n + sid * seg, seg)]
            )

    return kern(src3, dst3, a, b)


def _newton_rsqrt(v):
    i = plsc.bitcast(v, jnp.int32)
    i = 0x5F3759DF - lax.shift_right_logical(i, 1)
    g = plsc.bitcast(i, jnp.float32)
    for _ in range(3):
        g = g * (1.5 - 0.5 * v * g * g)
    return g


def _sc_scatter(src3, dst3, ew3, xl2, degp):
    _, rpw, ch = src3.shape
    n2, dh = xl2.shape
    n = n2 // 2
    stage_rows = 64
    parts = _row_partition(n, _NS, stage_rows)
    mesh = plsc.VectorSubcoreMesh(core_axis_name="c", subcore_axis_name="s")

    @functools.partial(
        pl.kernel,
        out_type=(
            jax.ShapeDtypeStruct((_NC, n, dh), jnp.float32),
            jax.ShapeDtypeStruct((n,), jnp.float32),
        ),
        mesh=mesh,
        scratch_types=(
            pltpu.VMEM((rpw, ch), jnp.int32),
            pltpu.VMEM((rpw, ch), jnp.int32),
            pltpu.VMEM((rpw, ch), jnp.float32),
            pltpu.VMEM((3, ch, dh), jnp.float32),
            pltpu.VMEM((64, dh), jnp.float32),
            pltpu.VMEM((n,), jnp.float32),
            pltpu.VMEM((2, 640), jnp.float32),
            pltpu.VMEM_SHARED((n, dh), jnp.float32),
            pltpu.SemaphoreType.DMA,
            pltpu.SemaphoreType.DMA,
        ),
        compiler_params=pltpu.CompilerParams(
            needs_layout_passes=False, use_tc_tiling_on_sc=False
        ),
    )
    def kern(src_hbm, dst_hbm, ew_hbm, xl_hbm, degp_hbm, msg_hbm, dis_hbm,
             src_v, dst_v, ew_v, rows_v, stage_v, dis_v, p_v,
             acc_sh, gsem, ssem):
        cid = lax.axis_index("c")
        sid = lax.axis_index("s")

        dbase = sid * 640
        last = n - 640 * (_NS - 1)

        def dgrp(g):
            deg = 1.0 + p_v[0, pl.ds(g * 16, 16)] + p_v[1, pl.ds(g * 16, 16)]
            dis_v[pl.ds(g * 16, 16)] = _newton_rsqrt(deg)

        @pl.when(sid < _NS - 1)
        def _():
            pltpu.sync_copy(degp_hbm.at[pl.ds(dbase, 640)], p_v.at[0])
            pltpu.sync_copy(degp_hbm.at[pl.ds(n + dbase, 640)], p_v.at[1])
            pl.loop(0, 640 // 16)(dgrp)
            pltpu.sync_copy(
                dis_v.at[pl.ds(0, 640)], dis_hbm.at[pl.ds(dbase, 640)]
            )

        @pl.when(sid == _NS - 1)
        def _():
            pltpu.sync_copy(
                degp_hbm.at[pl.ds(dbase, last)], p_v.at[0, pl.ds(0, last)]
            )
            pltpu.sync_copy(
                degp_hbm.at[pl.ds(n + dbase, last)], p_v.at[1, pl.ds(0, last)]
            )
            pl.loop(0, last // 16)(dgrp)
            pltpu.sync_copy(
                dis_v.at[pl.ds(0, last)], dis_hbm.at[pl.ds(dbase, last)]
            )

        def zrow(t):
            for j in range(dh // 16):
                stage_v[t, pl.ds(j * 16, 16)] = jnp.zeros((16,), jnp.float32)

        pl.loop(0, 64)(zrow)
        for t, (base, sizes) in enumerate(parts):
            @pl.when(sid == t)
            def _(base=base, sizes=sizes):
                off = 0
                for sz in sizes:
                    pltpu.sync_copy(
                        stage_v.at[pl.ds(0, sz)],
                        acc_sh.at[pl.ds(base + off, sz)],
                    )
                    off += sz

        pltpu.sync_copy(src_hbm.at[sid], src_v)
        pltpu.sync_copy(dst_hbm.at[sid], dst_v)
        pltpu.sync_copy(ew_hbm.at[sid], ew_v)

        plsc.subcore_barrier()
        pltpu.sync_copy(dis_hbm, dis_v)

        def remap(r_):
            for g in range(ch // 16):
                sv = src_v[r_, pl.ds(g * 16, 16)]
                dg = plsc.load_gather(dis_v, [sv])
                ew_v[r_, pl.ds(g * 16, 16)] = ew_v[r_, pl.ds(g * 16, 16)] * dg
                src_v[r_, pl.ds(g * 16, 16)] = sv * 2 + cid

        pl.loop(0, rpw)(remap)

        def fire_gather(rr, buf):
            @pl.when(rr < rpw)
            def _():
                pltpu.async_copy(xl_hbm.at[src_v.at[rr]], rows_v.at[buf], gsem)

        def wait_gather(rr, buf):
            pltpu.make_async_copy(
                xl_hbm.at[src_v.at[rr]], rows_v.at[buf], gsem
            ).wait()

        def fire_scatter(rr, buf):
            pltpu.async_copy(
                rows_v.at[buf], acc_sh.at[dst_v.at[rr]], ssem, add=True
            )

        def wait_scatter(rr, buf):
            pltpu.make_async_copy(
                rows_v.at[buf], acc_sh.at[dst_v.at[rr]], ssem
            ).wait()

        def scale(rr, buf):
            def sgroup(g):
                ewg = ew_v[rr, pl.ds(g * 16, 16)]
                for l in range(16):
                    sv = jnp.full((16,), ewg[l], jnp.float32)
                    e = g * 16 + l
                    for j in range(dh // 16):
                        rows_v[buf, e, pl.ds(j * 16, 16)] = (
                            rows_v[buf, e, pl.ds(j * 16, 16)] * sv
                        )

            pl.loop(0, ch // 16)(sgroup)

        fire_gather(0, 0)
        fire_gather(1, 1)
        wait_gather(0, 0)
        scale(0, 0)
        fire_scatter(0, 0)
        fire_gather(2, 2)

        def ring(r0):
            for q in range(3):
                rr = r0 + q
                buf = (1 + q) % 3
                wait_gather(rr, buf)
                scale(rr, buf)
                wait_scatter(rr - 1, (buf + 2) % 3)
                fire_scatter(rr, buf)
                fire_gather(rr + 2, (buf + 2) % 3)

        tail0 = 1 + 3 * ((rpw - 4) // 3)
        pl.loop(1, tail0, step=3)(ring)
        for rr in range(tail0, rpw):
            wait_gather(rr, rr % 3)
            scale(rr, rr % 3)
            wait_scatter(rr - 1, (rr - 1) % 3)
            fire_scatter(rr, rr % 3)
            fire_gather(rr + 2, (rr + 2) % 3)
        wait_scatter(rpw - 1, (rpw - 1) % 3)

        plsc.subcore_barrier()
        for t, (base, sizes) in enumerate(parts):
            @pl.when(sid == t)
            def _(base=base, sizes=sizes):
                off = 0
                for sz in sizes:
                    pltpu.sync_copy(
                        acc_sh.at[pl.ds(base + off, sz)],
                        stage_v.at[pl.ds(0, sz)],
                    )
                    pltpu.sync_copy(
                        stage_v.at[pl.ds(0, sz)],
                        msg_hbm.at[cid, pl.ds(base + off, sz)],
                    )
                    off += sz

    return kern(src3, dst3, ew3, xl2, degp)


def _tc_combine(degp_t, msg, x_lin, bias):
    n, d = x_lin.shape
    blk = 400

    def body(p_ref, m_ref, xl_ref, b_ref, o_ref):
        deg = 1.0 + p_ref[:, 0:1] + p_ref[:, 1:2]
        dis = lax.rsqrt(deg)
        m_full = jnp.concatenate([m_ref[0], m_ref[1]], axis=1)
        o_ref[...] = dis * m_full + (dis * dis) * xl_ref[...] + b_ref[...]

    return pl.pallas_call(
        body,
        grid=(n // blk,),
        in_specs=[
            pl.BlockSpec((blk, 2), lambda i: (i, 0)),
            pl.BlockSpec((2, blk, d // 2), lambda i: (0, i, 0)),
            pl.BlockSpec((blk, d), lambda i: (i, 0)),
            pl.BlockSpec((1, d), lambda i: (0, 0)),
        ],
        out_specs=pl.BlockSpec((blk, d), lambda i: (i, 0)),
        out_shape=jax.ShapeDtypeStruct((n, d), jnp.float32),
    )(degp_t, msg, x_lin, bias)


def kernel(x, edge_index, W_pred, b_pred, W_gcn, b_gcn):
    n, d = x.shape
    e = edge_index.shape[1]
    rpw = e // (_NW * _CH)
    rpw2 = e // (_NS * _CH)

    src = edge_index[0].astype(jnp.int32)
    dst = edge_index[1].astype(jnp.int32)
    src3 = src.reshape(_NW, rpw, _CH)
    dst3 = dst.reshape(_NW, rpw, _CH)

    w_ext = jnp.concatenate(
        [W_gcn, W_pred[:d], W_pred[d:], jnp.zeros((d, d - 2), jnp.float32)],
        axis=1,
    )
    bias_row = jnp.zeros((1, 2 * d), jnp.float32).at[0, d + 1].set(b_pred[0])

    xe = _tc_matmul(x, w_ext, bias_row)
    x_lin = xe[:, :d]
    a = xe[:, d]
    b = xe[:, d + 1]

    ew3, degp = _sc_edge_weights(src3, dst3, a, b)
    degp_t = degp.reshape(_NC, n).T

    msg, _ = _sc_scatter(
        src.reshape(_NS, rpw2, _CH),
        dst.reshape(_NS, rpw2, _CH),
        ew3.reshape(_NS, rpw2, _CH),
        x_lin.reshape(2 * n, d // 2),
        degp,
    )
    out = _tc_combine(degp_t, msg, x_lin, b_gcn.reshape(1, d))
    return out
